# baseline (speedup 1.0000x reference)
import jax
import jax.numpy as jnp
from jax.experimental import pallas as pl
from jax.experimental.pallas import tpu as pltpu

VOCAB_ = 32000
EMB_ = 1024
HID_ = 1024
BATCH_ = 64
SEQ_ = 512

T_BLK = 8
TOK_BLK = T_BLK * BATCH_
N_BLK = SEQ_ // T_BLK


def _fused_kernel(src_ref, emb_ref, wi_ref, wh_ref, b_ref, out_ref,
                  gbuf0, gbuf1, gbuf2, xbuf, h_ref, c_ref, sems):
    j = pl.program_id(0)
    nblk = pl.num_programs(0)
    bufs = (gbuf0, gbuf1, gbuf2)

    def issue_range(base, slot, lo, hi):
        buf = bufs[slot]
        for mi in range(lo, hi):
            tok = src_ref[base + mi]
            pltpu.make_async_copy(
                emb_ref.at[tok], buf.at[mi], sems.at[slot]
            ).start()

    def issue(base, slot):
        issue_range(base, slot, 0, TOK_BLK)

    @pl.when(j == 0)
    def _():
        h_ref[...] = jnp.zeros_like(h_ref)
        c_ref[...] = jnp.zeros_like(c_ref)
        issue(0, 0)
        issue(TOK_BLK, 1)

    nxt_base = jnp.minimum(j + 2, nblk - 1) * TOK_BLK

    def step(slot):
        buf = bufs[slot]
        pltpu.make_async_copy(buf, buf, sems.at[slot]).wait()
        issue_range(nxt_base, (slot + 2) % 3, 0, TOK_BLK // 2)
        xbuf[...] = (
            jax.lax.dot_general(
                buf[...], wi_ref[...],
                dimension_numbers=(((1,), (1,)), ((), ())),
                preferred_element_type=jnp.float32,
            )
            + b_ref[...]
        )

    @pl.when(jax.lax.rem(j, 3) == 0)
    def _():
        step(0)

    @pl.when(jax.lax.rem(j, 3) == 1)
    def _():
        step(1)

    @pl.when(jax.lax.rem(j, 3) == 2)
    def _():
        step(2)

    def sig(v):
        return 0.5 * jnp.tanh(0.5 * v) + 0.5

    h = h_ref[...]
    c = c_ref[...]
    for k in range(T_BLK):
        gates = xbuf[pl.ds(k * BATCH_, BATCH_)] + jnp.dot(
            h, wh_ref[...], preferred_element_type=jnp.float32
        )
        i_g = sig(gates[:, :HID_])
        f_g = sig(gates[:, HID_:2 * HID_])
        g_g = jnp.tanh(gates[:, 2 * HID_:3 * HID_])
        o_g = sig(gates[:, 3 * HID_:])
        c = f_g * c + i_g * g_g
        h = o_g * jnp.tanh(c)
    h_ref[...] = h
    c_ref[...] = c

    @pl.when(jax.lax.rem(j, 3) == 0)
    def _():
        issue_range(nxt_base, 2, TOK_BLK // 2, TOK_BLK)

    @pl.when(jax.lax.rem(j, 3) == 1)
    def _():
        issue_range(nxt_base, 0, TOK_BLK // 2, TOK_BLK)

    @pl.when(jax.lax.rem(j, 3) == 2)
    def _():
        issue_range(nxt_base, 1, TOK_BLK // 2, TOK_BLK)

    @pl.when(j == nblk - 1)
    def _():
        out_ref[0] = h
        out_ref[1] = c
        a = N_BLK % 3
        b = (N_BLK + 1) % 3
        pltpu.make_async_copy(bufs[a], bufs[a], sems.at[a]).wait()
        pltpu.make_async_copy(bufs[b], bufs[b], sems.at[b]).wait()


def _fused_call(src_flat, emb, w_ih, w_hhT, bias, *, interpret=False):
    return pl.pallas_call(
        _fused_kernel,
        out_shape=jax.ShapeDtypeStruct((2, BATCH_, HID_), jnp.float32),
        grid_spec=pltpu.PrefetchScalarGridSpec(
            num_scalar_prefetch=1,
            grid=(N_BLK,),
            in_specs=[
                pl.BlockSpec(memory_space=pl.ANY),
                pl.BlockSpec((4 * HID_, EMB_), lambda j, s: (0, 0)),
                pl.BlockSpec((HID_, 4 * HID_), lambda j, s: (0, 0)),
                pl.BlockSpec((1, 4 * HID_), lambda j, s: (0, 0)),
            ],
            out_specs=pl.BlockSpec((2, BATCH_, HID_), lambda j, s: (0, 0, 0)),
            scratch_shapes=[
                pltpu.VMEM((TOK_BLK, EMB_), jnp.float32),
                pltpu.VMEM((TOK_BLK, EMB_), jnp.float32),
                pltpu.VMEM((TOK_BLK, EMB_), jnp.float32),
                pltpu.VMEM((TOK_BLK, 4 * HID_), jnp.float32),
                pltpu.VMEM((BATCH_, HID_), jnp.float32),
                pltpu.VMEM((BATCH_, HID_), jnp.float32),
                pltpu.SemaphoreType.DMA((3,)),
            ],
        ),
        compiler_params=pltpu.CompilerParams(
            dimension_semantics=("arbitrary",),
            vmem_limit_bytes=58 * 1024 * 1024,
        ),
        name="lstm_fused",
        interpret=interpret,
    )(src_flat, emb, w_ih, w_hhT, bias)


def kernel(source, emb, W_ih, W_hh, b_ih, b_hh, *, interpret=False):
    src_flat = jnp.transpose(source).reshape(-1)
    w_hhT = jnp.transpose(W_hh)
    bias = (b_ih + b_hh).reshape(1, 4 * HID_)
    return _fused_call(src_flat, emb, W_ih, w_hhT, bias, interpret=interpret)


# speedup vs baseline: 1.0287x; 1.0287x over previous
import jax
import jax.numpy as jnp
from jax.experimental import pallas as pl
from jax.experimental.pallas import tpu as pltpu

VOCAB_ = 32000
EMB_ = 1024
HID_ = 1024
BATCH_ = 64
SEQ_ = 512

T_BLK = 8
TOK_BLK = T_BLK * BATCH_
N_BLK = SEQ_ // T_BLK


def _fused_kernel(src_ref, emb_ref, wi_ref, wh_ref, b_ref, out_ref,
                  gbuf0, gbuf1, gbuf2, xbuf, h_ref, c_ref, sems):
    j = pl.program_id(0)
    nblk = pl.num_programs(0)
    bufs = (gbuf0, gbuf1, gbuf2)

    def issue(base, slot):
        buf = bufs[slot]
        for mi in range(TOK_BLK):
            tok = src_ref[base + mi]
            pltpu.make_async_copy(
                emb_ref.at[tok], buf.at[mi], sems.at[slot]
            ).start()

    @pl.when(j == 0)
    def _():
        h_ref[...] = jnp.zeros_like(h_ref)
        c_ref[...] = jnp.zeros_like(c_ref)
        issue(0, 0)
        issue(TOK_BLK, 1)

    nxt_base = jnp.minimum(j + 2, nblk - 1) * TOK_BLK

    def step(slot):
        buf = bufs[slot]
        pltpu.make_async_copy(buf, buf, sems.at[slot]).wait()
        issue(nxt_base, (slot + 2) % 3)
        xbuf[...] = (
            jax.lax.dot_general(
                buf[...], wi_ref[...],
                dimension_numbers=(((1,), (1,)), ((), ())),
                preferred_element_type=jnp.float32,
            )
            + b_ref[...]
        )

    @pl.when(jax.lax.rem(j, 3) == 0)
    def _():
        step(0)

    @pl.when(jax.lax.rem(j, 3) == 1)
    def _():
        step(1)

    @pl.when(jax.lax.rem(j, 3) == 2)
    def _():
        step(2)

    def sig(v):
        return 0.5 * jnp.tanh(0.5 * v) + 0.5

    h = h_ref[...]
    c = c_ref[...]
    for k in range(T_BLK):
        gates = xbuf[pl.ds(k * BATCH_, BATCH_)] + jnp.dot(
            h, wh_ref[...], preferred_element_type=jnp.float32
        )
        i_g = sig(gates[:, :HID_])
        f_g = sig(gates[:, HID_:2 * HID_])
        g_g = jnp.tanh(gates[:, 2 * HID_:3 * HID_])
        o_g = sig(gates[:, 3 * HID_:])
        c = f_g * c + i_g * g_g
        h = o_g * jnp.tanh(c)
    h_ref[...] = h
    c_ref[...] = c

    @pl.when(j == nblk - 1)
    def _():
        out_ref[0] = h
        out_ref[1] = c
        a = N_BLK % 3
        b = (N_BLK + 1) % 3
        pltpu.make_async_copy(bufs[a], bufs[a], sems.at[a]).wait()
        pltpu.make_async_copy(bufs[b], bufs[b], sems.at[b]).wait()


def _fused_call(src_flat, emb, w_ih, w_hhT, bias, *, interpret=False):
    return pl.pallas_call(
        _fused_kernel,
        out_shape=jax.ShapeDtypeStruct((2, BATCH_, HID_), jnp.float32),
        grid_spec=pltpu.PrefetchScalarGridSpec(
            num_scalar_prefetch=1,
            grid=(N_BLK,),
            in_specs=[
                pl.BlockSpec(memory_space=pl.ANY),
                pl.BlockSpec((4 * HID_, EMB_), lambda j, s: (0, 0)),
                pl.BlockSpec((HID_, 4 * HID_), lambda j, s: (0, 0)),
                pl.BlockSpec((1, 4 * HID_), lambda j, s: (0, 0)),
            ],
            out_specs=pl.BlockSpec((2, BATCH_, HID_), lambda j, s: (0, 0, 0)),
            scratch_shapes=[
                pltpu.VMEM((TOK_BLK, EMB_), jnp.float32),
                pltpu.VMEM((TOK_BLK, EMB_), jnp.float32),
                pltpu.VMEM((TOK_BLK, EMB_), jnp.float32),
                pltpu.VMEM((TOK_BLK, 4 * HID_), jnp.float32),
                pltpu.VMEM((BATCH_, HID_), jnp.float32),
                pltpu.VMEM((BATCH_, HID_), jnp.float32),
                pltpu.SemaphoreType.DMA((3,)),
            ],
        ),
        compiler_params=pltpu.CompilerParams(
            dimension_semantics=("arbitrary",),
            vmem_limit_bytes=58 * 1024 * 1024,
        ),
        name="lstm_fused",
        interpret=interpret,
    )(src_flat, emb, w_ih, w_hhT, bias)


def kernel(source, emb, W_ih, W_hh, b_ih, b_hh, *, interpret=False):
    src_flat = jnp.transpose(source).reshape(-1)
    w_hhT = jnp.transpose(W_hh)
    bias = (b_ih + b_hh).reshape(1, 4 * HID_)
    return _fused_call(src_flat, emb, W_ih, w_hhT, bias, interpret=interpret)


# speedup vs baseline: 1.0318x; 1.0030x over previous
import jax
import jax.numpy as jnp
from jax.experimental import pallas as pl
from jax.experimental.pallas import tpu as pltpu

VOCAB_ = 32000
EMB_ = 1024
HID_ = 1024
BATCH_ = 64
SEQ_ = 512

T_BLK = 8
TOK_BLK = T_BLK * BATCH_
N_BLK = SEQ_ // T_BLK


def _fused_kernel(src_ref, emb_ref, wi_ref, wh_ref, b_ref, out_ref,
                  gbuf0, gbuf1, gbuf2, xbuf, h_ref, c_ref, sems):
    j = pl.program_id(0)
    nblk = pl.num_programs(0)
    bufs = (gbuf0, gbuf1, gbuf2)

    def issue(base, slot):
        buf = bufs[slot]
        for mi in range(TOK_BLK):
            tok = src_ref[base + mi]
            pltpu.make_async_copy(
                emb_ref.at[tok], buf.at[mi], sems.at[slot]
            ).start()

    @pl.when(j == 0)
    def _():
        h_ref[...] = jnp.zeros_like(h_ref)
        c_ref[...] = jnp.zeros_like(c_ref)
        issue(0, 0)
        issue(TOK_BLK, 1)

    nxt_base = jnp.minimum(j + 2, nblk - 1) * TOK_BLK

    def step(slot):
        buf = bufs[slot]
        pltpu.make_async_copy(buf, buf, sems.at[slot]).wait()
        issue(nxt_base, (slot + 2) % 3)
        xbuf[...] = (
            jax.lax.dot_general(
                buf[...], wi_ref[...],
                dimension_numbers=(((1,), (1,)), ((), ())),
                preferred_element_type=jnp.float32,
            )
            + b_ref[...]
        )

    @pl.when(jax.lax.rem(j, 3) == 0)
    def _():
        step(0)

    @pl.when(jax.lax.rem(j, 3) == 1)
    def _():
        step(1)

    @pl.when(jax.lax.rem(j, 3) == 2)
    def _():
        step(2)

    def sig(v):
        return 0.5 * jnp.tanh(0.5 * v) + 0.5

    h = h_ref[...]
    c = c_ref[...]
    for k in range(T_BLK):
        gates = xbuf[pl.ds(k * BATCH_, BATCH_)] + jnp.dot(
            h, wh_ref[...], preferred_element_type=jnp.float32
        )
        i_g = sig(gates[:, :HID_])
        f_g = sig(gates[:, HID_:2 * HID_])
        g_g = jnp.tanh(gates[:, 2 * HID_:3 * HID_])
        o_g = sig(gates[:, 3 * HID_:])
        c = f_g * c + i_g * g_g
        h = o_g * jnp.tanh(c)
    h_ref[...] = h
    c_ref[...] = c

    @pl.when(j == nblk - 1)
    def _():
        out_ref[0] = h
        out_ref[1] = c
        a = N_BLK % 3
        b = (N_BLK + 1) % 3
        pltpu.make_async_copy(bufs[a], bufs[a], sems.at[a]).wait()
        pltpu.make_async_copy(bufs[b], bufs[b], sems.at[b]).wait()


def _fused_call(src_flat, emb, w_ih, w_hhT, bias):
    return pl.pallas_call(
        _fused_kernel,
        out_shape=jax.ShapeDtypeStruct((2, BATCH_, HID_), jnp.float32),
        grid_spec=pltpu.PrefetchScalarGridSpec(
            num_scalar_prefetch=1,
            grid=(N_BLK,),
            in_specs=[
                pl.BlockSpec(memory_space=pl.ANY),
                pl.BlockSpec((4 * HID_, EMB_), lambda j, s: (0, 0)),
                pl.BlockSpec((HID_, 4 * HID_), lambda j, s: (0, 0)),
                pl.BlockSpec((1, 4 * HID_), lambda j, s: (0, 0)),
            ],
            out_specs=pl.BlockSpec((2, BATCH_, HID_), lambda j, s: (0, 0, 0)),
            scratch_shapes=[
                pltpu.VMEM((TOK_BLK, EMB_), jnp.float32),
                pltpu.VMEM((TOK_BLK, EMB_), jnp.float32),
                pltpu.VMEM((TOK_BLK, EMB_), jnp.float32),
                pltpu.VMEM((TOK_BLK, 4 * HID_), jnp.float32),
                pltpu.VMEM((BATCH_, HID_), jnp.float32),
                pltpu.VMEM((BATCH_, HID_), jnp.float32),
                pltpu.SemaphoreType.DMA((3,)),
            ],
        ),
        compiler_params=pltpu.CompilerParams(
            dimension_semantics=("arbitrary",),
            vmem_limit_bytes=58 * 1024 * 1024,
        ),
        name="lstm_fused",
    )(src_flat, emb, w_ih, w_hhT, bias)


def kernel(source, emb, W_ih, W_hh, b_ih, b_hh):
    src_flat = jnp.transpose(source).reshape(-1)
    w_hhT = jnp.transpose(W_hh)
    bias = (b_ih + b_hh).reshape(1, 4 * HID_)
    return _fused_call(src_flat, emb, W_ih, w_hhT, bias)
